# baseline (speedup 1.0000x reference)
import jax
import jax.numpy as jnp
from jax import lax
from jax.experimental import pallas as pl
from jax.experimental.pallas import tpu as pltpu
from jax.experimental.pallas import tpu_sc as plsc

N_NODES = 10000
N_EDGES = 160000
LAT = 128
EPS = 1e-5

NC = 2
NS = 16
NW = NC * NS
GC = 128
GGC = 128
E_PAD = 163840
E_HALF = E_PAD // 2
N_GCH = E_HALF // GGC
AGG_PAD = 10240
ROWS_PT = AGG_PAD // NS
N_SCH = E_HALF // GC
SC_CH = N_SCH // NC
SCPT = SC_CH // NS
SCPT_PAD = 24

ET = 4096
NT = 2000

_SC_MESH = plsc.VectorSubcoreMesh(core_axis_name="c", subcore_axis_name="s",
                                  num_cores=NC, num_subcores=NS)


CPT = N_GCH // NS
NBUF = 2
TAB_PT = AGG_PAD // NS


def _split_gather_body(ta_hbm, tb_hbm, si_hbm, ri_hbm, out_hbm,
                       tab_sh, idx, rows, semg, semo):
    c = lax.axis_index("c")
    s = lax.axis_index("s")
    c0 = pl.multiple_of(s * CPT, 8)

    def stage(tab, ib):
        nst = TAB_PT // GGC

        def soff(k):
            return pl.multiple_of(s * TAB_PT + k * GGC, GGC)

        pltpu.async_copy(tab.at[pl.ds(soff(0), GGC)], rows.at[0], semg)
        for k in range(nst):
            pltpu.make_async_copy(tab.at[pl.ds(0, GGC)], rows.at[k % 2],
                                  semg).wait()
            pltpu.async_copy(rows.at[k % 2], tab_sh.at[pl.ds(soff(k), GGC)],
                             semo)
            if k + 1 < nst:
                if k >= 1:
                    pltpu.make_async_copy(tab.at[pl.ds(0, GGC)],
                                          rows.at[(k + 1) % 2], semo).wait()
                pltpu.async_copy(tab.at[pl.ds(soff(k + 1), GGC)],
                                 rows.at[(k + 1) % 2], semg)
        pltpu.sync_copy(ib.at[pl.ds(c0, CPT)], idx)
        pltpu.make_async_copy(tab.at[pl.ds(0, GGC)], rows.at[0], semo).wait()
        pltpu.make_async_copy(tab.at[pl.ds(0, GGC)], rows.at[1], semo).wait()

    @pl.when(c == 0)
    def _():
        stage(ta_hbm, si_hbm)

    @pl.when(c == 1)
    def _():
        stage(tb_hbm, ri_hbm)

    plsc.subcore_barrier()
    out = out_hbm.at[c]

    def start(t, b):
        pltpu.async_copy(tab_sh.at[idx.at[t]], rows.at[b], semg)

    def wait_g(b):
        pltpu.make_async_copy(out.at[pl.ds(0, GGC)], rows.at[b], semg).wait()

    def wait_o(b):
        pltpu.make_async_copy(out.at[pl.ds(0, GGC)], rows.at[b], semo).wait()

    for t in range(NBUF - 1):
        start(t, t)

    def step(t2, carry):
        for b in range(NBUF):
            t = t2 * NBUF + b

            @pl.when(t >= 1)
            def _():
                wait_o((b + NBUF - 1) % NBUF)

            @pl.when(t + NBUF - 1 < CPT)
            def _():
                start(t + NBUF - 1, (b + NBUF - 1) % NBUF)

            wait_g(b)
            off = pl.multiple_of((c0 + t) * GGC, GGC)
            pltpu.async_copy(rows.at[b], out.at[pl.ds(off, GGC)], semo)
        return carry

    lax.fori_loop(0, CPT // NBUF, step, 0)
    wait_o((CPT - 1) % NBUF)


_split_gather = pl.kernel(
    _split_gather_body,
    out_type=jax.ShapeDtypeStruct((NC, E_HALF, LAT), jnp.float32),
    mesh=_SC_MESH,
    scratch_types=[
        pltpu.VMEM_SHARED((AGG_PAD, LAT), jnp.float32),
        pltpu.VMEM((CPT, GGC), jnp.int32),
        pltpu.VMEM((NBUF, GGC, LAT), jnp.float32),
        pltpu.SemaphoreType.DMA,
        pltpu.SemaphoreType.DMA,
    ],
)


def _agg_body(e_hbm, ri_hbm, out_hbm, agg_sh, ridx, rows, semr, sems):
    c = lax.axis_index("c")
    s = lax.axis_index("s")

    def zrow(i, carry):
        for k in range(LAT // 16):
            rows[0, i, pl.ds(k * 16, 16)] = jnp.zeros((16,), jnp.float32)
        return carry

    lax.fori_loop(0, GC, zrow, 0)
    for k in range(ROWS_PT // GC):
        off = pl.multiple_of(s * ROWS_PT + k * GC, GC)
        pltpu.async_copy(rows.at[0], agg_sh.at[pl.ds(off, GC)], sems)

    c0 = c * SC_CH + s * SCPT
    pltpu.sync_copy(ri_hbm.at[c, s], ridx)
    for k in range(ROWS_PT // GC):
        pltpu.make_async_copy(e_hbm.at[pl.ds(0, GC)], rows.at[0], sems).wait()
    plsc.subcore_barrier()

    def start(t, b):
        off = pl.multiple_of((c0 + t) * GC, GC)
        pltpu.async_copy(e_hbm.at[pl.ds(off, GC)], rows.at[b], semr)

    def wait_r(b):
        pltpu.make_async_copy(e_hbm.at[pl.ds(0, GC)], rows.at[b], semr).wait()

    def wait_s(b):
        pltpu.make_async_copy(e_hbm.at[pl.ds(0, GC)], rows.at[b], sems).wait()

    start(0, 0)

    def step(t2, carry):
        for b in range(2):
            t = t2 * 2 + b

            @pl.when(t + 1 < SCPT)
            def _():
                start(t + 1, 1 - b)

            wait_r(b)

            @pl.when(t >= 2)
            def _():
                wait_s(b)

            pltpu.async_copy(rows.at[b], agg_sh.at[ridx.at[t]], sems, add=True)
        return carry

    lax.fori_loop(0, SCPT // 2, step, 0)
    wait_s(0)
    wait_s(1)
    plsc.subcore_barrier()

    nst = ROWS_PT // GC

    def doff(k):
        return pl.multiple_of(s * ROWS_PT + k * GC, GC)

    pltpu.async_copy(agg_sh.at[pl.ds(doff(0), GC)], rows.at[0], semr)
    for k in range(nst):
        pltpu.make_async_copy(e_hbm.at[pl.ds(0, GC)], rows.at[k % 2],
                              semr).wait()
        pltpu.async_copy(rows.at[k % 2], out_hbm.at[c].at[pl.ds(doff(k), GC)],
                         sems)
        if k + 1 < nst:
            if k >= 1:
                pltpu.make_async_copy(e_hbm.at[pl.ds(0, GC)],
                                      rows.at[(k + 1) % 2], sems).wait()
            pltpu.async_copy(agg_sh.at[pl.ds(doff(k + 1), GC)],
                             rows.at[(k + 1) % 2], semr)
    pltpu.make_async_copy(e_hbm.at[pl.ds(0, GC)], rows.at[0], sems).wait()
    pltpu.make_async_copy(e_hbm.at[pl.ds(0, GC)], rows.at[1], sems).wait()


_agg_call = pl.kernel(
    _agg_body,
    out_type=jax.ShapeDtypeStruct((NC, AGG_PAD, LAT), jnp.float32),
    mesh=_SC_MESH,
    scratch_types=[
        pltpu.VMEM_SHARED((AGG_PAD, LAT), jnp.float32),
        pltpu.VMEM((SCPT_PAD, GC), jnp.int32),
        pltpu.VMEM((2, GC, LAT), jnp.float32),
        pltpu.SemaphoreType.DMA,
        pltpu.SemaphoreType.DMA,
    ],
)


def _ln(h, g, beta):
    mu = jnp.mean(h, axis=-1, keepdims=True)
    var = jnp.mean((h - mu) ** 2, axis=-1, keepdims=True)
    return (h - mu) * lax.rsqrt(var + EPS) * g + beta


def _enc_node_body(x_ref, w1_ref, b1_ref, w2_ref, b2_ref, w3_ref, b3_ref,
                   g_ref, beta_ref, ws_ref, wr_ref, out_ref, ps_ref, pr_ref):
    h = jax.nn.relu(jnp.dot(x_ref[:], w1_ref[:],
                            preferred_element_type=jnp.float32) + b1_ref[:])
    h = jax.nn.relu(jnp.dot(h, w2_ref[:],
                            preferred_element_type=jnp.float32) + b2_ref[:])
    h = jnp.dot(h, w3_ref[:], preferred_element_type=jnp.float32) + b3_ref[:]
    nout = _ln(h, g_ref[:], beta_ref[:])
    out_ref[:] = nout
    ps_ref[:] = jnp.dot(nout, ws_ref[:], preferred_element_type=jnp.float32)
    pr_ref[:] = jnp.dot(nout, wr_ref[:], preferred_element_type=jnp.float32)


def _enc_edge_body(xs_ref, xr_ref, w1_ref, wn_ref, wm_ref, b1_ref, w2_ref,
                   b2_ref, w3_ref, b3_ref, g_ref, beta_ref, out_ref):
    x = xs_ref[:] - xr_ref[:]
    nw = jnp.sqrt(jnp.sum(x[:, 0:3] * x[:, 0:3], axis=-1, keepdims=True))
    nm = jnp.sqrt(jnp.sum(x[:, 3:5] * x[:, 3:5], axis=-1, keepdims=True))
    h = jax.nn.relu(jnp.dot(x, w1_ref[:], preferred_element_type=jnp.float32)
                    + nw * wn_ref[:] + nm * wm_ref[:] + b1_ref[:])
    h = jax.nn.relu(jnp.dot(h, w2_ref[:],
                            preferred_element_type=jnp.float32) + b2_ref[:])
    h = jnp.dot(h, w3_ref[:], preferred_element_type=jnp.float32) + b3_ref[:]
    out_ref[:] = _ln(h, g_ref[:], beta_ref[:])


def _edge_blk_body(e_ref, ms_ref, mr_ref, w1_ref, b1_ref, w2_ref, b2_ref,
                   w3_ref, b3_ref, g_ref, beta_ref, enew_ref, eout_ref):
    e = e_ref[:]
    h = jax.nn.relu(jnp.dot(e, w1_ref[:],
                            preferred_element_type=jnp.float32)
                    + ms_ref[:] + mr_ref[:] + b1_ref[:])
    h = jax.nn.relu(jnp.dot(h, w2_ref[:],
                            preferred_element_type=jnp.float32) + b2_ref[:])
    h = jnp.dot(h, w3_ref[:], preferred_element_type=jnp.float32) + b3_ref[:]
    h = _ln(h, g_ref[:], beta_ref[:])
    enew_ref[:] = h
    eout_ref[:] = e + h


def _node_blk_body(n_ref, agg0_ref, agg1_ref, agg2_ref, agg3_ref,
                   w1n_ref, w1a_ref, b1_ref,
                   w2_ref, b2_ref, w3_ref, b3_ref, g_ref, beta_ref,
                   ws_ref, wr_ref, nout_ref, ps_ref, pr_ref):
    n = n_ref[:]
    agg = (agg0_ref[:] + agg1_ref[:]) + (agg2_ref[:] + agg3_ref[:])
    h = jax.nn.relu(jnp.dot(n, w1n_ref[:], preferred_element_type=jnp.float32)
                    + jnp.dot(agg, w1a_ref[:],
                              preferred_element_type=jnp.float32)
                    + b1_ref[:])
    h = jax.nn.relu(jnp.dot(h, w2_ref[:],
                            preferred_element_type=jnp.float32) + b2_ref[:])
    h = jnp.dot(h, w3_ref[:], preferred_element_type=jnp.float32) + b3_ref[:]
    nout = n + _ln(h, g_ref[:], beta_ref[:])
    nout_ref[:] = nout
    ps_ref[:] = jnp.dot(nout, ws_ref[:], preferred_element_type=jnp.float32)
    pr_ref[:] = jnp.dot(nout, wr_ref[:], preferred_element_type=jnp.float32)


def _dec_body(n_ref, w1_ref, b1_ref, w2_ref, b2_ref, w3_ref, b3_ref, out_ref):
    h = jax.nn.relu(jnp.dot(n_ref[:], w1_ref[:],
                            preferred_element_type=jnp.float32) + b1_ref[:])
    h = jax.nn.relu(jnp.dot(h, w2_ref[:],
                            preferred_element_type=jnp.float32) + b2_ref[:])
    out_ref[:] = (jnp.dot(h, w3_ref[:], preferred_element_type=jnp.float32)
                  + b3_ref[:])


def _full(shape):
    return pl.BlockSpec(shape, lambda i: (0,) * len(shape))


def _rows(t, d):
    return pl.BlockSpec((t, d), lambda i: (i, 0))


def _enc_node_call(x, w1, b1, w2, b2, w3, b3, g, beta, ws, wr):
    return pl.pallas_call(
        _enc_node_body,
        grid=(N_NODES // NT,),
        in_specs=[_rows(NT, x.shape[1]), _full(w1.shape), _full(b1.shape),
                  _full(w2.shape), _full(b2.shape), _full(w3.shape),
                  _full(b3.shape), _full(g.shape), _full(beta.shape),
                  _full(ws.shape), _full(wr.shape)],
        out_specs=[_rows(NT, LAT)] * 3,
        out_shape=[jax.ShapeDtypeStruct((N_NODES, LAT), jnp.float32),
                   jax.ShapeDtypeStruct((AGG_PAD, LAT), jnp.float32),
                   jax.ShapeDtypeStruct((AGG_PAD, LAT), jnp.float32)],
    )(x, w1, b1, w2, b2, w3, b3, g, beta, ws, wr)


def _enc_edge_call(xs, xr, w1, wn, wm, b1, w2, b2, w3, b3, g, beta):
    return pl.pallas_call(
        _enc_edge_body,
        grid=(xs.shape[0] // ET,),
        in_specs=[_rows(ET, LAT), _rows(ET, LAT), _full(w1.shape),
                  _full(wn.shape), _full(wm.shape), _full(b1.shape),
                  _full(w2.shape), _full(b2.shape), _full(w3.shape),
                  _full(b3.shape), _full(g.shape), _full(beta.shape)],
        out_specs=_rows(ET, LAT),
        out_shape=jax.ShapeDtypeStruct((xs.shape[0], LAT), jnp.float32),
    )(xs, xr, w1, wn, wm, b1, w2, b2, w3, b3, g, beta)


def _edge_blk_call(e, ms, mr, w1, b1, w2, b2, w3, b3, g, beta):
    return pl.pallas_call(
        _edge_blk_body,
        grid=(e.shape[0] // ET,),
        in_specs=[_rows(ET, LAT), _rows(ET, LAT), _rows(ET, LAT),
                  _full(w1.shape), _full(b1.shape), _full(w2.shape),
                  _full(b2.shape), _full(w3.shape), _full(b3.shape),
                  _full(g.shape), _full(beta.shape)],
        out_specs=[_rows(ET, LAT), _rows(ET, LAT)],
        out_shape=[jax.ShapeDtypeStruct((e.shape[0], LAT), jnp.float32),
                   jax.ShapeDtypeStruct((e.shape[0], LAT), jnp.float32)],
    )(e, ms, mr, w1, b1, w2, b2, w3, b3, g, beta)


def _node_blk_call(n, agg0, agg1, agg2, agg3, w1n, w1a, b1, w2, b2, w3, b3,
                   g, beta, ws, wr):
    return pl.pallas_call(
        _node_blk_body,
        grid=(N_NODES // NT,),
        in_specs=[_rows(NT, LAT), _rows(NT, LAT), _rows(NT, LAT),
                  _rows(NT, LAT), _rows(NT, LAT),
                  _full(w1n.shape), _full(w1a.shape), _full(b1.shape),
                  _full(w2.shape), _full(b2.shape), _full(w3.shape),
                  _full(b3.shape), _full(g.shape), _full(beta.shape),
                  _full(ws.shape), _full(wr.shape)],
        out_specs=[_rows(NT, LAT)] * 3,
        out_shape=[jax.ShapeDtypeStruct((N_NODES, LAT), jnp.float32),
                   jax.ShapeDtypeStruct((AGG_PAD, LAT), jnp.float32),
                   jax.ShapeDtypeStruct((AGG_PAD, LAT), jnp.float32)],
    )(n, agg0, agg1, agg2, agg3, w1n, w1a, b1, w2, b2, w3, b3, g, beta,
      ws, wr)


def _dec_call(n, w1, b1, w2, b2, w3, b3):
    return pl.pallas_call(
        _dec_body,
        grid=(N_NODES // NT,),
        in_specs=[_rows(NT, LAT), _full(w1.shape), _full(b1.shape),
                  _full(w2.shape), _full(b2.shape), _full(w3.shape),
                  _full(b3.shape)],
        out_specs=_rows(NT, LAT),
        out_shape=jax.ShapeDtypeStruct((N_NODES, LAT), jnp.float32),
    )(n, w1, b1, w2, b2, w3, b3)


def kernel(world_pos, prev_world_pos, mesh_pos, params, node_type, edge_index):
    senders = edge_index[0]
    receivers = edge_index[1]
    pad = E_PAD - N_EDGES

    s_g = jnp.concatenate([senders, jnp.zeros((pad,), jnp.int32)]
                          ).reshape(2 * N_GCH, GGC)
    r_g = jnp.concatenate([receivers, jnp.zeros((pad,), jnp.int32)]
                          ).reshape(2 * N_GCH, GGC)
    r_s = jnp.concatenate([receivers,
                           jnp.full((pad,), N_NODES, jnp.int32)]
                          ).reshape(2 * N_SCH, GC)

    velocity = world_pos - prev_world_pos
    nt_onehot = jax.nn.one_hot(node_type, 9, dtype=jnp.float32)
    node_feat = jnp.concatenate([velocity, nt_onehot], axis=-1)
    node_feat = (node_feat - params['node_mean']) / params['node_std']
    node_feat = jnp.pad(node_feat, ((0, 0), (0, 4)))

    pos = jnp.zeros((AGG_PAD, LAT), jnp.float32)
    pos = pos.at[:N_NODES, 0:3].set(world_pos).at[:N_NODES, 3:5].set(mesh_pos)
    sA, sB = s_g[:N_GCH], s_g[N_GCH:]
    rA, rB = r_g[:N_GCH], r_g[N_GCH:]
    def scatter_idx(rh):
        return jnp.pad(rh.reshape(NC, NS, SCPT, GC),
                       ((0, 0), (0, 0), (0, SCPT_PAD - SCPT), (0, 0)))

    rsA = scatter_idx(r_s[:N_SCH])
    rsB = scatter_idx(r_s[N_SCH:])
    posgA = _split_gather(pos, pos, sA, rA)
    posgB = _split_gather(pos, pos, sB, rB)

    def row(b):
        return b.reshape(1, -1)

    be = params['blk_edge']
    bn = params['blk_node']
    w1e_all = be['W'][0][:, :LAT, :]
    w1s_all = be['W'][0][:, LAT:2 * LAT, :]
    w1r_all = be['W'][0][:, 2 * LAT:, :]
    w1n_all = bn['W'][0][:, :LAT, :]
    w1a_all = bn['W'][0][:, LAT:, :]

    pn = params['enc_node']
    w1 = jnp.pad(pn['W'][0], ((0, 4), (0, 0)))
    node_lat, ps, pr = _enc_node_call(
        node_feat, w1, row(pn['b'][0]), pn['W'][1], row(pn['b'][1]),
        pn['W'][2], row(pn['b'][2]), row(pn['g']), row(pn['beta']),
        w1s_all[0], w1r_all[0])

    pe = params['enc_edge']
    w1s = pe['W'][0] / params['edge_std'][:, None]
    b1p = pe['b'][0] - (params['edge_mean'] / params['edge_std']) @ pe['W'][0]
    w1x = jnp.zeros((LAT, LAT), jnp.float32)
    w1x = w1x.at[0:3].set(w1s[0:3]).at[3:5].set(w1s[4:6])
    enc_args = (w1x, row(w1s[3]), row(w1s[6]), row(b1p), pe['W'][1],
                row(pe['b'][1]), pe['W'][2], row(pe['b'][2]), row(pe['g']),
                row(pe['beta']))
    eA = _enc_edge_call(posgA[0], posgA[1], *enc_args)
    eB = _enc_edge_call(posgB[0], posgB[1], *enc_args)

    for i in range(15):
        edge_args = (w1e_all[i], row(be['b'][0][i]), be['W'][1][i],
                     row(be['b'][1][i]), be['W'][2][i], row(be['b'][2][i]),
                     row(be['g'][i]), row(be['beta'][i]))
        msgA = _split_gather(ps, pr, sA, rA)
        neA, eA = _edge_blk_call(eA, msgA[0], msgA[1], *edge_args)
        aggA = _agg_call(neA, rsA)
        msgB = _split_gather(ps, pr, sB, rB)
        neB, eB = _edge_blk_call(eB, msgB[0], msgB[1], *edge_args)
        aggB = _agg_call(neB, rsB)
        j = min(i + 1, 14)
        node_lat, ps, pr = _node_blk_call(
            node_lat, aggA[0], aggA[1], aggB[0], aggB[1],
            w1n_all[i], w1a_all[i], row(bn['b'][0][i]),
            bn['W'][1][i], row(bn['b'][1][i]), bn['W'][2][i],
            row(bn['b'][2][i]), row(bn['g'][i]), row(bn['beta'][i]),
            w1s_all[j], w1r_all[j])

    pd = params['dec']
    w3 = jnp.pad(pd['W'][2], ((0, 0), (0, LAT - 3)))
    b3 = jnp.pad(pd['b'][2], (0, LAT - 3))
    pred = _dec_call(node_lat, pd['W'][0], row(pd['b'][0]), pd['W'][1],
                     row(pd['b'][1]), w3, row(b3))[:, :3]

    acc = pred * params['out_std'] + params['out_mean']
    pred_pos = 2.0 * world_pos + acc - prev_world_pos
    mask = (node_type == 0)[:, None]
    new_world = jnp.where(mask, pred_pos, world_pos)
    new_prev = jnp.where(mask, world_pos, prev_world_pos)
    return new_world, new_prev


# speedup vs baseline: 1.0101x; 1.0101x over previous
import jax
import jax.numpy as jnp
from jax import lax
from jax.experimental import pallas as pl
from jax.experimental.pallas import tpu as pltpu
from jax.experimental.pallas import tpu_sc as plsc

N_NODES = 10000
N_EDGES = 160000
LAT = 128
EPS = 1e-5

NC = 2
NS = 16
NW = NC * NS
GC = 128
GGC = 128
E_PAD = 163840
N_GCH = E_PAD // GGC
AGG_PAD = 10240
ROWS_PT = AGG_PAD // NS
N_SCH = E_PAD // GC
SC_CH = N_SCH // NC
SCPT = SC_CH // NS

ET = 4096
NT = 2000

_SC_MESH = plsc.VectorSubcoreMesh(core_axis_name="c", subcore_axis_name="s",
                                  num_cores=NC, num_subcores=NS)


CPT = N_GCH // NS
NBUF = 2
TAB_PT = AGG_PAD // NS


def _split_gather_body(ta_hbm, tb_hbm, si_hbm, ri_hbm, out_hbm,
                       tab_sh, idx, rows, semg, semo):
    c = lax.axis_index("c")
    s = lax.axis_index("s")
    c0 = pl.multiple_of(s * CPT, 8)

    def stage(tab, ib):
        nst = TAB_PT // GGC

        def soff(k):
            return pl.multiple_of(s * TAB_PT + k * GGC, GGC)

        pltpu.async_copy(tab.at[pl.ds(soff(0), GGC)], rows.at[0], semg)
        for k in range(nst):
            pltpu.make_async_copy(tab.at[pl.ds(0, GGC)], rows.at[k % 2],
                                  semg).wait()
            pltpu.async_copy(rows.at[k % 2], tab_sh.at[pl.ds(soff(k), GGC)],
                             semo)
            if k + 1 < nst:
                if k >= 1:
                    pltpu.make_async_copy(tab.at[pl.ds(0, GGC)],
                                          rows.at[(k + 1) % 2], semo).wait()
                pltpu.async_copy(tab.at[pl.ds(soff(k + 1), GGC)],
                                 rows.at[(k + 1) % 2], semg)
        pltpu.sync_copy(ib.at[pl.ds(c0, CPT)], idx)
        pltpu.make_async_copy(tab.at[pl.ds(0, GGC)], rows.at[0], semo).wait()
        pltpu.make_async_copy(tab.at[pl.ds(0, GGC)], rows.at[1], semo).wait()

    @pl.when(c == 0)
    def _():
        stage(ta_hbm, si_hbm)

    @pl.when(c == 1)
    def _():
        stage(tb_hbm, ri_hbm)

    plsc.subcore_barrier()
    out = out_hbm.at[c]

    def start(t, b):
        pltpu.async_copy(tab_sh.at[idx.at[t]], rows.at[b], semg)

    def wait_g(b):
        pltpu.make_async_copy(out.at[pl.ds(0, GGC)], rows.at[b], semg).wait()

    def wait_o(b):
        pltpu.make_async_copy(out.at[pl.ds(0, GGC)], rows.at[b], semo).wait()

    for t in range(NBUF - 1):
        start(t, t)

    def step(t2, carry):
        for b in range(NBUF):
            t = t2 * NBUF + b

            @pl.when(t >= 1)
            def _():
                wait_o((b + NBUF - 1) % NBUF)

            @pl.when(t + NBUF - 1 < CPT)
            def _():
                start(t + NBUF - 1, (b + NBUF - 1) % NBUF)

            wait_g(b)
            off = pl.multiple_of((c0 + t) * GGC, GGC)
            pltpu.async_copy(rows.at[b], out.at[pl.ds(off, GGC)], semo)
        return carry

    lax.fori_loop(0, CPT // NBUF, step, 0)
    wait_o((CPT - 1) % NBUF)


_split_gather = pl.kernel(
    _split_gather_body,
    out_type=jax.ShapeDtypeStruct((NC, E_PAD, LAT), jnp.float32),
    mesh=_SC_MESH,
    scratch_types=[
        pltpu.VMEM_SHARED((AGG_PAD, LAT), jnp.float32),
        pltpu.VMEM((CPT, GGC), jnp.int32),
        pltpu.VMEM((NBUF, GGC, LAT), jnp.float32),
        pltpu.SemaphoreType.DMA,
        pltpu.SemaphoreType.DMA,
    ],
)


def _agg_body(e_hbm, ri_hbm, out_hbm, agg_sh, ridx, rows, semr, sems):
    c = lax.axis_index("c")
    s = lax.axis_index("s")

    def zrow(i, carry):
        for k in range(LAT // 16):
            rows[0, i, pl.ds(k * 16, 16)] = jnp.zeros((16,), jnp.float32)
        return carry

    lax.fori_loop(0, GC, zrow, 0)
    for k in range(ROWS_PT // GC):
        off = pl.multiple_of(s * ROWS_PT + k * GC, GC)
        pltpu.async_copy(rows.at[0], agg_sh.at[pl.ds(off, GC)], sems)

    c0 = pl.multiple_of(c * SC_CH + s * SCPT, 8)
    pltpu.sync_copy(ri_hbm.at[pl.ds(c0, SCPT)], ridx)
    for k in range(ROWS_PT // GC):
        pltpu.make_async_copy(e_hbm.at[pl.ds(0, GC)], rows.at[0], sems).wait()
    plsc.subcore_barrier()

    def start(t, b):
        off = pl.multiple_of((c0 + t) * GC, GC)
        pltpu.async_copy(e_hbm.at[pl.ds(off, GC)], rows.at[b], semr)

    def wait_r(b):
        pltpu.make_async_copy(e_hbm.at[pl.ds(0, GC)], rows.at[b], semr).wait()

    def wait_s(b):
        pltpu.make_async_copy(e_hbm.at[pl.ds(0, GC)], rows.at[b], sems).wait()

    start(0, 0)

    def step(t2, carry):
        for b in range(2):
            t = t2 * 2 + b

            @pl.when(t + 1 < SCPT)
            def _():
                start(t + 1, 1 - b)

            wait_r(b)

            @pl.when(t >= 2)
            def _():
                wait_s(b)

            pltpu.async_copy(rows.at[b], agg_sh.at[ridx.at[t]], sems, add=True)
        return carry

    lax.fori_loop(0, SCPT // 2, step, 0)
    wait_s(0)
    wait_s(1)
    plsc.subcore_barrier()

    nst = ROWS_PT // GC

    def doff(k):
        return pl.multiple_of(s * ROWS_PT + k * GC, GC)

    pltpu.async_copy(agg_sh.at[pl.ds(doff(0), GC)], rows.at[0], semr)
    for k in range(nst):
        pltpu.make_async_copy(e_hbm.at[pl.ds(0, GC)], rows.at[k % 2],
                              semr).wait()
        pltpu.async_copy(rows.at[k % 2], out_hbm.at[c].at[pl.ds(doff(k), GC)],
                         sems)
        if k + 1 < nst:
            if k >= 1:
                pltpu.make_async_copy(e_hbm.at[pl.ds(0, GC)],
                                      rows.at[(k + 1) % 2], sems).wait()
            pltpu.async_copy(agg_sh.at[pl.ds(doff(k + 1), GC)],
                             rows.at[(k + 1) % 2], semr)
    pltpu.make_async_copy(e_hbm.at[pl.ds(0, GC)], rows.at[0], sems).wait()
    pltpu.make_async_copy(e_hbm.at[pl.ds(0, GC)], rows.at[1], sems).wait()


_agg_call = pl.kernel(
    _agg_body,
    out_type=jax.ShapeDtypeStruct((NC, AGG_PAD, LAT), jnp.float32),
    mesh=_SC_MESH,
    scratch_types=[
        pltpu.VMEM_SHARED((AGG_PAD, LAT), jnp.float32),
        pltpu.VMEM((SCPT, GC), jnp.int32),
        pltpu.VMEM((2, GC, LAT), jnp.float32),
        pltpu.SemaphoreType.DMA,
        pltpu.SemaphoreType.DMA,
    ],
)


def _ln(h, g, beta):
    mu = jnp.mean(h, axis=-1, keepdims=True)
    var = jnp.mean((h - mu) ** 2, axis=-1, keepdims=True)
    return (h - mu) * lax.rsqrt(var + EPS) * g + beta


def _enc_node_body(x_ref, w1_ref, b1_ref, w2_ref, b2_ref, w3_ref, b3_ref,
                   g_ref, beta_ref, ws_ref, wr_ref, out_ref, ps_ref, pr_ref):
    h = jax.nn.relu(jnp.dot(x_ref[:], w1_ref[:],
                            preferred_element_type=jnp.float32) + b1_ref[:])
    h = jax.nn.relu(jnp.dot(h, w2_ref[:],
                            preferred_element_type=jnp.float32) + b2_ref[:])
    h = jnp.dot(h, w3_ref[:], preferred_element_type=jnp.float32) + b3_ref[:]
    nout = _ln(h, g_ref[:], beta_ref[:])
    out_ref[:] = nout
    ps_ref[:] = jnp.dot(nout, ws_ref[:], preferred_element_type=jnp.float32)
    pr_ref[:] = jnp.dot(nout, wr_ref[:], preferred_element_type=jnp.float32)


def _enc_edge_body(xs_ref, xr_ref, w1_ref, wn_ref, wm_ref, b1_ref, w2_ref,
                   b2_ref, w3_ref, b3_ref, g_ref, beta_ref, out_ref):
    x = xs_ref[:] - xr_ref[:]
    nw = jnp.sqrt(jnp.sum(x[:, 0:3] * x[:, 0:3], axis=-1, keepdims=True))
    nm = jnp.sqrt(jnp.sum(x[:, 3:5] * x[:, 3:5], axis=-1, keepdims=True))
    h = jax.nn.relu(jnp.dot(x, w1_ref[:], preferred_element_type=jnp.float32)
                    + nw * wn_ref[:] + nm * wm_ref[:] + b1_ref[:])
    h = jax.nn.relu(jnp.dot(h, w2_ref[:],
                            preferred_element_type=jnp.float32) + b2_ref[:])
    h = jnp.dot(h, w3_ref[:], preferred_element_type=jnp.float32) + b3_ref[:]
    out_ref[:] = _ln(h, g_ref[:], beta_ref[:])


def _edge_blk_body(e_ref, ms_ref, mr_ref, w1_ref, b1_ref, w2_ref, b2_ref,
                   w3_ref, b3_ref, g_ref, beta_ref, enew_ref, eout_ref):
    e = e_ref[:]
    h = jax.nn.relu(jnp.dot(e, w1_ref[:],
                            preferred_element_type=jnp.float32)
                    + ms_ref[:] + mr_ref[:] + b1_ref[:])
    h = jax.nn.relu(jnp.dot(h, w2_ref[:],
                            preferred_element_type=jnp.float32) + b2_ref[:])
    h = jnp.dot(h, w3_ref[:], preferred_element_type=jnp.float32) + b3_ref[:]
    h = _ln(h, g_ref[:], beta_ref[:])
    enew_ref[:] = h
    eout_ref[:] = e + h


def _node_blk_body(n_ref, agg0_ref, agg1_ref, w1n_ref, w1a_ref, b1_ref,
                   w2_ref, b2_ref, w3_ref, b3_ref, g_ref, beta_ref,
                   ws_ref, wr_ref, nout_ref, ps_ref, pr_ref):
    n = n_ref[:]
    h = jax.nn.relu(jnp.dot(n, w1n_ref[:], preferred_element_type=jnp.float32)
                    + jnp.dot(agg0_ref[:] + agg1_ref[:], w1a_ref[:],
                              preferred_element_type=jnp.float32)
                    + b1_ref[:])
    h = jax.nn.relu(jnp.dot(h, w2_ref[:],
                            preferred_element_type=jnp.float32) + b2_ref[:])
    h = jnp.dot(h, w3_ref[:], preferred_element_type=jnp.float32) + b3_ref[:]
    nout = n + _ln(h, g_ref[:], beta_ref[:])
    nout_ref[:] = nout
    ps_ref[:] = jnp.dot(nout, ws_ref[:], preferred_element_type=jnp.float32)
    pr_ref[:] = jnp.dot(nout, wr_ref[:], preferred_element_type=jnp.float32)


def _dec_body(n_ref, w1_ref, b1_ref, w2_ref, b2_ref, w3_ref, b3_ref, out_ref):
    h = jax.nn.relu(jnp.dot(n_ref[:], w1_ref[:],
                            preferred_element_type=jnp.float32) + b1_ref[:])
    h = jax.nn.relu(jnp.dot(h, w2_ref[:],
                            preferred_element_type=jnp.float32) + b2_ref[:])
    out_ref[:] = (jnp.dot(h, w3_ref[:], preferred_element_type=jnp.float32)
                  + b3_ref[:])


def _full(shape):
    return pl.BlockSpec(shape, lambda i: (0,) * len(shape))


def _rows(t, d):
    return pl.BlockSpec((t, d), lambda i: (i, 0))


def _enc_node_call(x, w1, b1, w2, b2, w3, b3, g, beta, ws, wr):
    return pl.pallas_call(
        _enc_node_body,
        grid=(N_NODES // NT,),
        in_specs=[_rows(NT, x.shape[1]), _full(w1.shape), _full(b1.shape),
                  _full(w2.shape), _full(b2.shape), _full(w3.shape),
                  _full(b3.shape), _full(g.shape), _full(beta.shape),
                  _full(ws.shape), _full(wr.shape)],
        out_specs=[_rows(NT, LAT)] * 3,
        out_shape=[jax.ShapeDtypeStruct((N_NODES, LAT), jnp.float32),
                   jax.ShapeDtypeStruct((AGG_PAD, LAT), jnp.float32),
                   jax.ShapeDtypeStruct((AGG_PAD, LAT), jnp.float32)],
    )(x, w1, b1, w2, b2, w3, b3, g, beta, ws, wr)


def _enc_edge_call(xs, xr, w1, wn, wm, b1, w2, b2, w3, b3, g, beta):
    return pl.pallas_call(
        _enc_edge_body,
        grid=(E_PAD // ET,),
        in_specs=[_rows(ET, LAT), _rows(ET, LAT), _full(w1.shape),
                  _full(wn.shape), _full(wm.shape), _full(b1.shape),
                  _full(w2.shape), _full(b2.shape), _full(w3.shape),
                  _full(b3.shape), _full(g.shape), _full(beta.shape)],
        out_specs=_rows(ET, LAT),
        out_shape=jax.ShapeDtypeStruct((E_PAD, LAT), jnp.float32),
    )(xs, xr, w1, wn, wm, b1, w2, b2, w3, b3, g, beta)


def _edge_blk_call(e, ms, mr, w1, b1, w2, b2, w3, b3, g, beta):
    return pl.pallas_call(
        _edge_blk_body,
        grid=(E_PAD // ET,),
        in_specs=[_rows(ET, LAT), _rows(ET, LAT), _rows(ET, LAT),
                  _full(w1.shape), _full(b1.shape), _full(w2.shape),
                  _full(b2.shape), _full(w3.shape), _full(b3.shape),
                  _full(g.shape), _full(beta.shape)],
        out_specs=[_rows(ET, LAT), _rows(ET, LAT)],
        out_shape=[jax.ShapeDtypeStruct((E_PAD, LAT), jnp.float32),
                   jax.ShapeDtypeStruct((E_PAD, LAT), jnp.float32)],
    )(e, ms, mr, w1, b1, w2, b2, w3, b3, g, beta)


def _node_blk_call(n, agg0, agg1, w1n, w1a, b1, w2, b2, w3, b3, g, beta,
                   ws, wr):
    return pl.pallas_call(
        _node_blk_body,
        grid=(N_NODES // NT,),
        in_specs=[_rows(NT, LAT), _rows(NT, LAT), _rows(NT, LAT),
                  _full(w1n.shape), _full(w1a.shape), _full(b1.shape),
                  _full(w2.shape), _full(b2.shape), _full(w3.shape),
                  _full(b3.shape), _full(g.shape), _full(beta.shape),
                  _full(ws.shape), _full(wr.shape)],
        out_specs=[_rows(NT, LAT)] * 3,
        out_shape=[jax.ShapeDtypeStruct((N_NODES, LAT), jnp.float32),
                   jax.ShapeDtypeStruct((AGG_PAD, LAT), jnp.float32),
                   jax.ShapeDtypeStruct((AGG_PAD, LAT), jnp.float32)],
    )(n, agg0, agg1, w1n, w1a, b1, w2, b2, w3, b3, g, beta, ws, wr)


def _dec_call(n, w1, b1, w2, b2, w3, b3):
    return pl.pallas_call(
        _dec_body,
        grid=(N_NODES // NT,),
        in_specs=[_rows(NT, LAT), _full(w1.shape), _full(b1.shape),
                  _full(w2.shape), _full(b2.shape), _full(w3.shape),
                  _full(b3.shape)],
        out_specs=_rows(NT, LAT),
        out_shape=jax.ShapeDtypeStruct((N_NODES, LAT), jnp.float32),
    )(n, w1, b1, w2, b2, w3, b3)


def kernel(world_pos, prev_world_pos, mesh_pos, params, node_type, edge_index):
    senders = edge_index[0]
    receivers = edge_index[1]
    pad = E_PAD - N_EDGES

    s_g = jnp.concatenate([senders, jnp.zeros((pad,), jnp.int32)]
                          ).reshape(N_GCH, GGC)
    r_g = jnp.concatenate([receivers, jnp.zeros((pad,), jnp.int32)]
                          ).reshape(N_GCH, GGC)
    r_s = jnp.concatenate([receivers,
                           jnp.full((pad,), N_NODES, jnp.int32)]
                          ).reshape(N_SCH, GC)

    velocity = world_pos - prev_world_pos
    nt_onehot = jax.nn.one_hot(node_type, 9, dtype=jnp.float32)
    node_feat = jnp.concatenate([velocity, nt_onehot], axis=-1)
    node_feat = (node_feat - params['node_mean']) / params['node_std']
    node_feat = jnp.pad(node_feat, ((0, 0), (0, 4)))

    pos = jnp.zeros((AGG_PAD, LAT), jnp.float32)
    pos = pos.at[:N_NODES, 0:3].set(world_pos).at[:N_NODES, 3:5].set(mesh_pos)
    posg = _split_gather(pos, pos, s_g, r_g)

    def row(b):
        return b.reshape(1, -1)

    be = params['blk_edge']
    bn = params['blk_node']
    w1e_all = be['W'][0][:, :LAT, :]
    w1s_all = be['W'][0][:, LAT:2 * LAT, :]
    w1r_all = be['W'][0][:, 2 * LAT:, :]
    w1n_all = bn['W'][0][:, :LAT, :]
    w1a_all = bn['W'][0][:, LAT:, :]

    pn = params['enc_node']
    w1 = jnp.pad(pn['W'][0], ((0, 4), (0, 0)))
    node_lat, ps, pr = _enc_node_call(
        node_feat, w1, row(pn['b'][0]), pn['W'][1], row(pn['b'][1]),
        pn['W'][2], row(pn['b'][2]), row(pn['g']), row(pn['beta']),
        w1s_all[0], w1r_all[0])

    pe = params['enc_edge']
    w1s = pe['W'][0] / params['edge_std'][:, None]
    b1p = pe['b'][0] - (params['edge_mean'] / params['edge_std']) @ pe['W'][0]
    w1x = jnp.zeros((LAT, LAT), jnp.float32)
    w1x = w1x.at[0:3].set(w1s[0:3]).at[3:5].set(w1s[4:6])
    edge_lat = _enc_edge_call(posg[0], posg[1], w1x, row(w1s[3]), row(w1s[6]),
                              row(b1p), pe['W'][1], row(pe['b'][1]),
                              pe['W'][2], row(pe['b'][2]), row(pe['g']),
                              row(pe['beta']))

    for i in range(15):
        msg = _split_gather(ps, pr, s_g, r_g)
        new_e, edge_lat = _edge_blk_call(
            edge_lat, msg[0], msg[1], w1e_all[i], row(be['b'][0][i]),
            be['W'][1][i],
            row(be['b'][1][i]), be['W'][2][i], row(be['b'][2][i]),
            row(be['g'][i]), row(be['beta'][i]))
        agg = _agg_call(new_e, r_s)
        j = min(i + 1, 14)
        node_lat, ps, pr = _node_blk_call(
            node_lat, agg[0], agg[1], w1n_all[i], w1a_all[i],
            row(bn['b'][0][i]),
            bn['W'][1][i], row(bn['b'][1][i]), bn['W'][2][i],
            row(bn['b'][2][i]), row(bn['g'][i]), row(bn['beta'][i]),
            w1s_all[j], w1r_all[j])

    pd = params['dec']
    w3 = jnp.pad(pd['W'][2], ((0, 0), (0, LAT - 3)))
    b3 = jnp.pad(pd['b'][2], (0, LAT - 3))
    pred = _dec_call(node_lat, pd['W'][0], row(pd['b'][0]), pd['W'][1],
                     row(pd['b'][1]), w3, row(b3))[:, :3]

    acc = pred * params['out_std'] + params['out_mean']
    pred_pos = 2.0 * world_pos + acc - prev_world_pos
    mask = (node_type == 0)[:, None]
    new_world = jnp.where(mask, pred_pos, world_pos)
    new_prev = jnp.where(mask, world_pos, prev_world_pos)
    return new_world, new_prev
